# baseline (speedup 1.0000x reference)
import numpy as np

import jax
import jax.numpy as jnp
from jax.experimental import pallas as pl
from jax.experimental.pallas import tpu as pltpu

_NB = 10


def _xw_kernel(x_ref, w_ref, o_ref):
    o_ref[...] = jnp.dot(x_ref[...], w_ref[...],
                         preferred_element_type=jnp.float32)


def _build_schedule(nb):
    rows, cols, fin_s2, ph1, wout, rstart = [], [], [], [], [], []
    for r in range(nb):
        order = list(range(r + 1, nb)) + list(range(0, r + 1))
        for j, c in enumerate(order):
            rows.append(r)
            cols.append(c)
            rstart.append(1 if j == 0 else 0)
            diag = c == r
            fin_s2.append(1 if diag else 0)
            ph1.append(1 if (c < r or diag) else 0)
            wout.append(1 if (diag and r == nb - 1) else 0)
    for r in range(nb - 1):
        for c in range(r + 1, nb):
            rows.append(r)
            cols.append(c)
            rstart.append(0)
            fin_s2.append(0)
            ph1.append(1)
            wout.append(1 if c == nb - 1 else 0)
    t_total = len(rows)
    rout = [0] * t_total
    nxt = nb - 1
    for t in range(t_total - 1, -1, -1):
        if wout[t]:
            nxt = rows[t]
        rout[t] = nxt
    mk = lambda a: np.asarray(a, dtype=np.int32)
    return (mk(rows), mk(cols), mk(rout), mk(rstart), mk(fin_s2), mk(ph1),
            mk(wout))


_SCHED = _build_schedule(_NB)


def _gcn_kernel(rows_ref, cols_ref, rout_ref, rstart_ref, fin_ref, ph1_ref,
                wout_ref, adj_ref, s1_ref, b1_ref, w2_ref, b2_ref, o_ref,
                h_ref, s2_ref, oacc_ref):
    t = pl.program_id(0)
    b = adj_ref.shape[0]
    r = rows_ref[t]
    c = cols_ref[t]
    adj_tile = adj_ref[:, 0, 0, :]

    @pl.when(t == 0)
    def _init():
        oacc_ref[...] = jnp.zeros_like(oacc_ref)

    @pl.when(rstart_ref[t] == 1)
    def _row_start():
        h_ref[...] = jnp.zeros_like(h_ref)

    in_pass_a = t < _NB * _NB

    @pl.when(in_pass_a)
    def _phase0():
        h_ref[...] += jnp.dot(adj_tile, s1_ref[pl.ds(c * b, b), :],
                              preferred_element_type=jnp.float32)

    @pl.when(fin_ref[t] == 1)
    def _finalize_s2():
        h = jnp.maximum(h_ref[...] + b1_ref[...], 0.0)
        s2_ref[pl.ds(r * b, b), :] = jnp.dot(
            h, w2_ref[...], preferred_element_type=jnp.float32)

    @pl.when(ph1_ref[t] == 1)
    def _phase1():
        oacc_ref[pl.ds(r * b, b), :] += jnp.dot(
            adj_tile, s2_ref[pl.ds(c * b, b), :],
            preferred_element_type=jnp.float32)

    @pl.when(wout_ref[t] == 1)
    def _writeout():
        o = oacc_ref[pl.ds(r * b, b), :] + b2_ref[...]
        m = jnp.max(o, axis=-1, keepdims=True)
        e = o - m
        lse = jnp.log(jnp.sum(jnp.exp(e), axis=-1, keepdims=True))
        o_ref[...] = e - lse


def kernel(x, adj, W1, b1, W2, b2):
    n, _ = x.shape
    hid = W1.shape[1]
    out_f = W2.shape[1]
    bsz = n // _NB

    s1 = pl.pallas_call(
        _xw_kernel,
        out_shape=jax.ShapeDtypeStruct((n, hid), jnp.float32),
    )(x, W1)

    b1r = b1.reshape(1, hid)
    b2r = b2.reshape(1, out_f)
    sched = tuple(jnp.asarray(a) for a in _SCHED)
    t_total = _SCHED[0].shape[0]

    grid_spec = pltpu.PrefetchScalarGridSpec(
        num_scalar_prefetch=7,
        grid=(t_total,),
        in_specs=[
            pl.BlockSpec((bsz, 1, 1, bsz),
                         lambda t, rows, cols, *_: (rows[t], cols[t], 0, 0)),
            pl.BlockSpec((n, hid), lambda t, *_: (0, 0)),
            pl.BlockSpec((1, hid), lambda t, *_: (0, 0)),
            pl.BlockSpec((hid, out_f), lambda t, *_: (0, 0)),
            pl.BlockSpec((1, out_f), lambda t, *_: (0, 0)),
        ],
        out_specs=pl.BlockSpec((bsz, out_f),
                               lambda t, rows, cols, rout, *_: (rout[t], 0)),
        scratch_shapes=[
            pltpu.VMEM((bsz, hid), jnp.float32),
            pltpu.VMEM((n, out_f), jnp.float32),
            pltpu.VMEM((n, out_f), jnp.float32),
        ],
    )

    adj4 = adj.reshape(n, _NB, 1, bsz)
    out = pl.pallas_call(
        _gcn_kernel,
        grid_spec=grid_spec,
        out_shape=jax.ShapeDtypeStruct((n, out_f), jnp.float32),
    )(*sched, adj4, s1, b1r, W2, b2r)
    return out


# speedup vs baseline: 11.5575x; 11.5575x over previous
import functools

import numpy as np

import jax
import jax.numpy as jnp
from jax.experimental import pallas as pl
from jax.experimental.pallas import tpu as pltpu

_NB = 10


def _build_schedule(nb):
    rows, cols, fin_s2, ph1, wout, rstart = [], [], [], [], [], []
    for r in range(nb):
        order = list(range(r + 1, nb)) + list(range(0, r + 1))
        for j, c in enumerate(order):
            rows.append(r)
            cols.append(c)
            rstart.append(1 if j == 0 else 0)
            diag = c == r
            fin_s2.append(1 if diag else 0)
            ph1.append(1 if (c < r or diag) else 0)
            wout.append(1 if (diag and r == nb - 1) else 0)
    for r in range(nb - 1):
        for c in range(r + 1, nb):
            rows.append(r)
            cols.append(c)
            rstart.append(0)
            fin_s2.append(0)
            ph1.append(1)
            wout.append(1 if c == nb - 1 else 0)
    t_total = len(rows)
    seen = set()
    ph1f = [0] * t_total
    for t in range(t_total):
        if ph1[t] and rows[t] not in seen:
            seen.add(rows[t])
            ph1f[t] = 1
    rout = [0] * t_total
    nxt = nb - 1
    for t in range(t_total - 1, -1, -1):
        if wout[t]:
            nxt = rows[t]
        rout[t] = nxt
    mk = lambda a: np.asarray(a, dtype=np.int32)
    return (mk(rows), mk(cols), mk(rout), mk(rstart), mk(fin_s2), mk(ph1),
            mk(ph1f), mk(wout))


_SCHED = _build_schedule(_NB)


def _xw_kernel(x_ref, w_ref, o_ref):
    n = x_ref.shape[0]
    o_ref[...] = jnp.zeros_like(o_ref)
    o_ref[pl.ds(0, n), :] = jnp.dot(x_ref[...], w_ref[...],
                                    preferred_element_type=jnp.float32)


def _gcn_kernel(n_valid, rows_ref, cols_ref, rout_ref, rstart_ref, fin_ref,
                ph1_ref, ph1f_ref, wout_ref, cmask_ref, adj_ref, s1_ref,
                b1_ref, w2_ref, b2_ref, o_ref, h_ref, s2_ref, oacc_ref):
    t = pl.program_id(0)
    b = adj_ref.shape[0]
    r = rows_ref[t]
    c = cols_ref[t]
    valid_last = n_valid - (_NB - 1) * b

    if valid_last < b:

        @pl.when(cmask_ref[t] == 1)
        def _zero_overhang_cols():
            adj_ref[:, pl.ds(valid_last, b - valid_last)] = jnp.zeros(
                (b, b - valid_last), jnp.float32)

    @pl.when(rstart_ref[t] == 1)
    def _row_start():
        h_ref[...] = jnp.zeros_like(h_ref)

    @pl.when(t < _NB * _NB)
    def _phase0():
        h_ref[...] += jnp.dot(adj_ref[...], s1_ref[pl.ds(c * b, b), :],
                              preferred_element_type=jnp.float32)

    @pl.when(fin_ref[t] == 1)
    def _finalize_s2():
        h = jnp.maximum(h_ref[...] + b1_ref[...], 0.0)
        s2_blk = jnp.dot(h, w2_ref[...], preferred_element_type=jnp.float32)
        row_ids = r * b + jax.lax.broadcasted_iota(jnp.int32, s2_blk.shape, 0)
        s2_ref[pl.ds(r * b, b), :] = jnp.where(row_ids < n_valid, s2_blk, 0.0)

    @pl.when(ph1_ref[t] == 1)
    def _phase1():
        contrib = jnp.dot(adj_ref[...], s2_ref[pl.ds(c * b, b), :],
                          preferred_element_type=jnp.float32)

        @pl.when(ph1f_ref[t] == 1)
        def _first():
            oacc_ref[pl.ds(r * b, b), :] = contrib

        @pl.when(ph1f_ref[t] == 0)
        def _rest():
            oacc_ref[pl.ds(r * b, b), :] += contrib

    @pl.when(wout_ref[t] == 1)
    def _writeout():
        o = oacc_ref[pl.ds(r * b, b), :] + b2_ref[...]
        m = jnp.max(o, axis=-1, keepdims=True)
        e = o - m
        lse = jnp.log(jnp.sum(jnp.exp(e), axis=-1, keepdims=True))
        o_ref[...] = e - lse


def kernel(x, adj, W1, b1, W2, b2):
    n, _ = x.shape
    hid = W1.shape[1]
    out_f = W2.shape[1]
    per_blk = (n + _NB - 1) // _NB
    bsz = ((per_blk + 127) // 128) * 128
    npad = _NB * bsz

    s1p = pl.pallas_call(
        _xw_kernel,
        out_shape=jax.ShapeDtypeStruct((npad, hid), jnp.float32),
    )(x, W1)

    b1r = b1.reshape(1, hid)
    b2r = b2.reshape(1, out_f)
    cmask = ((_SCHED[1] == _NB - 1) & (n % bsz != 0)).astype(np.int32)
    sched = tuple(jnp.asarray(a) for a in _SCHED) + (jnp.asarray(cmask),)
    t_total = _SCHED[0].shape[0]

    grid_spec = pltpu.PrefetchScalarGridSpec(
        num_scalar_prefetch=9,
        grid=(t_total,),
        in_specs=[
            pl.BlockSpec((bsz, bsz),
                         lambda t, rows, cols, *_: (rows[t], cols[t])),
            pl.BlockSpec((npad, hid), lambda t, *_: (0, 0)),
            pl.BlockSpec((1, hid), lambda t, *_: (0, 0)),
            pl.BlockSpec((hid, out_f), lambda t, *_: (0, 0)),
            pl.BlockSpec((1, out_f), lambda t, *_: (0, 0)),
        ],
        out_specs=pl.BlockSpec((bsz, out_f),
                               lambda t, rows, cols, rout, *_: (rout[t], 0)),
        scratch_shapes=[
            pltpu.VMEM((bsz, hid), jnp.float32),
            pltpu.VMEM((npad, out_f), jnp.float32),
            pltpu.VMEM((npad, out_f), jnp.float32),
        ],
    )

    out = pl.pallas_call(
        functools.partial(_gcn_kernel, n),
        grid_spec=grid_spec,
        out_shape=jax.ShapeDtypeStruct((n, out_f), jnp.float32),
    )(*sched, adj, s1p, b1r, W2, b2r)
    return out
